# baseline (speedup 1.0000x reference)
import jax
import jax.numpy as jnp
from jax.experimental import pallas as pl
from jax.experimental.pallas import tpu as pltpu


def _round_up(x, m):
    return ((x + m - 1) // m) * m


def _mlp_kernel(x_ref, w1_ref, b1_ref, w2_ref, b2_ref, ha32_ref, ha16_ref):
    h = jnp.dot(x_ref[...], w1_ref[...], preferred_element_type=jnp.float32)
    h = jnp.maximum(h + b1_ref[...], 0.0)
    ha = jnp.dot(h, w2_ref[...], preferred_element_type=jnp.float32) + b2_ref[...]
    ha32_ref[...] = ha
    ha16_ref[...] = ha.astype(jnp.bfloat16)


def _prop_kernel(adj_ref, ha_ref, hp_ref):
    a16 = adj_ref[...].astype(jnp.bfloat16)
    hp_ref[...] = jnp.dot(a16, ha_ref[...], preferred_element_type=jnp.float32)


def kernel(x, adj, w1, b1, w2, b2):
    N, n_in = x.shape
    h1 = w1.shape[0]
    h2 = w2.shape[0]
    f32 = jnp.float32

    h2_pad = _round_up(h2, 128)
    TM1 = 1024 if N % 1024 == 0 else 256
    TM2 = 256 if N % 256 == 0 else 128
    N_pad = _round_up(N, max(TM1, TM2))

    x_p = x.astype(f32)
    adj_p = adj.astype(f32)
    if N_pad != N:
        x_p = jnp.pad(x_p, ((0, N_pad - N), (0, 0)))
        adj_p = jnp.pad(adj_p, ((0, N_pad - N), (0, N_pad - N)))

    w1_t = w1.T.astype(f32)
    b1_r = b1.reshape(1, h1).astype(f32)
    w2_t = jnp.pad(w2.T.astype(f32), ((0, 0), (0, h2_pad - h2)))
    b2_r = jnp.pad(b2.reshape(1, h2).astype(f32), ((0, 0), (0, h2_pad - h2)))

    ha32, ha16 = pl.pallas_call(
        _mlp_kernel,
        out_shape=(
            jax.ShapeDtypeStruct((N_pad, h2_pad), f32),
            jax.ShapeDtypeStruct((N_pad, h2_pad), jnp.bfloat16),
        ),
        grid=(N_pad // TM1,),
        in_specs=[
            pl.BlockSpec((TM1, n_in), lambda i: (i, 0)),
            pl.BlockSpec((n_in, h1), lambda i: (0, 0)),
            pl.BlockSpec((1, h1), lambda i: (0, 0)),
            pl.BlockSpec((h1, h2_pad), lambda i: (0, 0)),
            pl.BlockSpec((1, h2_pad), lambda i: (0, 0)),
        ],
        out_specs=(
            pl.BlockSpec((TM1, h2_pad), lambda i: (i, 0)),
            pl.BlockSpec((TM1, h2_pad), lambda i: (i, 0)),
        ),
        compiler_params=pltpu.CompilerParams(
            dimension_semantics=("parallel",)),
    )(x_p, w1_t, b1_r, w2_t, b2_r)

    hp = pl.pallas_call(
        _prop_kernel,
        out_shape=jax.ShapeDtypeStruct((N_pad, h2_pad), f32),
        grid=(N_pad // TM2,),
        in_specs=[
            pl.BlockSpec((TM2, N_pad), lambda i: (i, 0)),
            pl.BlockSpec((N_pad, h2_pad), lambda i: (0, 0)),
        ],
        out_specs=pl.BlockSpec((TM2, h2_pad), lambda i: (i, 0)),
        compiler_params=pltpu.CompilerParams(
            dimension_semantics=("parallel",)),
        cost_estimate=pl.CostEstimate(
            flops=2 * N_pad * N_pad * h2_pad,
            transcendentals=0,
            bytes_accessed=4 * N_pad * N_pad + 2 * N_pad * h2_pad
                           + 4 * N_pad * h2_pad,
        ),
    )(adj_p, ha16)

    return ha32[:N, :h2], hp[:N, :h2]


# speedup vs baseline: 1.0518x; 1.0518x over previous
import jax
import jax.numpy as jnp
from jax.experimental import pallas as pl
from jax.experimental.pallas import tpu as pltpu


def _round_up(x, m):
    return ((x + m - 1) // m) * m


def _mlp_kernel(x_ref, w1_ref, b1_ref, w2_ref, b2_ref, ha32_ref, ha16_ref):
    h = jnp.dot(x_ref[...], w1_ref[...], preferred_element_type=jnp.float32)
    h = jnp.maximum(h + b1_ref[...], 0.0)
    ha = jnp.dot(h, w2_ref[...], preferred_element_type=jnp.float32) + b2_ref[...]
    ha32_ref[...] = ha
    ha16_ref[...] = ha.astype(jnp.bfloat16)


def _prop_kernel(adj_ref, ha_ref, hp_ref):
    a16 = adj_ref[...].astype(jnp.bfloat16)
    hp_ref[...] = jnp.dot(a16, ha_ref[...], preferred_element_type=jnp.float32)


def kernel(x, adj, w1, b1, w2, b2):
    N, n_in = x.shape
    h1 = w1.shape[0]
    h2 = w2.shape[0]
    f32 = jnp.float32

    h2_pad = _round_up(h2, 128)
    TM1 = 1024 if N % 1024 == 0 else 256
    TM2 = 1024 if N % 1024 == 0 else 256
    N_pad = _round_up(N, max(TM1, TM2))

    x_p = x.astype(f32)
    adj_p = adj.astype(f32)
    if N_pad != N:
        x_p = jnp.pad(x_p, ((0, N_pad - N), (0, 0)))
        adj_p = jnp.pad(adj_p, ((0, N_pad - N), (0, N_pad - N)))

    w1_t = w1.T.astype(f32)
    b1_r = b1.reshape(1, h1).astype(f32)
    w2_t = jnp.pad(w2.T.astype(f32), ((0, 0), (0, h2_pad - h2)))
    b2_r = jnp.pad(b2.reshape(1, h2).astype(f32), ((0, 0), (0, h2_pad - h2)))

    ha32, ha16 = pl.pallas_call(
        _mlp_kernel,
        out_shape=(
            jax.ShapeDtypeStruct((N_pad, h2_pad), f32),
            jax.ShapeDtypeStruct((N_pad, h2_pad), jnp.bfloat16),
        ),
        grid=(N_pad // TM1,),
        in_specs=[
            pl.BlockSpec((TM1, n_in), lambda i: (i, 0)),
            pl.BlockSpec((n_in, h1), lambda i: (0, 0)),
            pl.BlockSpec((1, h1), lambda i: (0, 0)),
            pl.BlockSpec((h1, h2_pad), lambda i: (0, 0)),
            pl.BlockSpec((1, h2_pad), lambda i: (0, 0)),
        ],
        out_specs=(
            pl.BlockSpec((TM1, h2_pad), lambda i: (i, 0)),
            pl.BlockSpec((TM1, h2_pad), lambda i: (i, 0)),
        ),
        compiler_params=pltpu.CompilerParams(
            dimension_semantics=("parallel",)),
    )(x_p, w1_t, b1_r, w2_t, b2_r)

    hp = pl.pallas_call(
        _prop_kernel,
        out_shape=jax.ShapeDtypeStruct((N_pad, h2_pad), f32),
        grid=(N_pad // TM2,),
        in_specs=[
            pl.BlockSpec((TM2, N_pad), lambda i: (i, 0)),
            pl.BlockSpec((N_pad, h2_pad), lambda i: (0, 0)),
        ],
        out_specs=pl.BlockSpec((TM2, h2_pad), lambda i: (i, 0)),
        compiler_params=pltpu.CompilerParams(
            dimension_semantics=("parallel",)),
        cost_estimate=pl.CostEstimate(
            flops=2 * N_pad * N_pad * h2_pad,
            transcendentals=0,
            bytes_accessed=4 * N_pad * N_pad + 2 * N_pad * h2_pad
                           + 4 * N_pad * h2_pad,
        ),
    )(adj_p, ha16)

    return ha32[:N, :h2], hp[:N, :h2]


# speedup vs baseline: 1.1155x; 1.0605x over previous
import jax
import jax.numpy as jnp
from jax.experimental import pallas as pl
from jax.experimental.pallas import tpu as pltpu


def _round_up(x, m):
    return ((x + m - 1) // m) * m


def _mlp_kernel(x_ref, w1_ref, b1_ref, w2_ref, b2_ref, ha32_ref, ha16_ref):
    h = jnp.dot(x_ref[...], w1_ref[...], preferred_element_type=jnp.float32)
    h = jnp.maximum(h + b1_ref[...], 0.0)
    ha = jnp.dot(h, w2_ref[...], preferred_element_type=jnp.float32) + b2_ref[...]
    ha32_ref[...] = ha
    ha16_ref[...] = ha.astype(jnp.bfloat16)


def _prop_kernel(adjl_ref, adjr_ref, ha_ref, hp_ref):
    half = ha_ref.shape[0] // 2
    al = adjl_ref[...].astype(jnp.bfloat16)
    ar = adjr_ref[...].astype(jnp.bfloat16)
    hp_ref[...] = (
        jnp.dot(al, ha_ref[:half], preferred_element_type=jnp.float32)
        + jnp.dot(ar, ha_ref[half:], preferred_element_type=jnp.float32)
    )


def kernel(x, adj, w1, b1, w2, b2):
    N, n_in = x.shape
    h1 = w1.shape[0]
    h2 = w2.shape[0]
    f32 = jnp.float32

    h2_pad = _round_up(h2, 128)
    TM1 = 1024 if N % 1024 == 0 else 256
    TM2 = 512 if N % 512 == 0 else 256
    N_pad = _round_up(N, max(TM1, TM2))

    x_p = x.astype(f32)
    adj_p = adj.astype(f32)
    if N_pad != N:
        x_p = jnp.pad(x_p, ((0, N_pad - N), (0, 0)))
        adj_p = jnp.pad(adj_p, ((0, N_pad - N), (0, N_pad - N)))

    w1_t = w1.T.astype(f32)
    b1_r = b1.reshape(1, h1).astype(f32)
    w2_t = jnp.pad(w2.T.astype(f32), ((0, 0), (0, h2_pad - h2)))
    b2_r = jnp.pad(b2.reshape(1, h2).astype(f32), ((0, 0), (0, h2_pad - h2)))

    ha32, ha16 = pl.pallas_call(
        _mlp_kernel,
        out_shape=(
            jax.ShapeDtypeStruct((N_pad, h2_pad), f32),
            jax.ShapeDtypeStruct((N_pad, h2_pad), jnp.bfloat16),
        ),
        grid=(N_pad // TM1,),
        in_specs=[
            pl.BlockSpec((TM1, n_in), lambda i: (i, 0)),
            pl.BlockSpec((n_in, h1), lambda i: (0, 0)),
            pl.BlockSpec((1, h1), lambda i: (0, 0)),
            pl.BlockSpec((h1, h2_pad), lambda i: (0, 0)),
            pl.BlockSpec((1, h2_pad), lambda i: (0, 0)),
        ],
        out_specs=(
            pl.BlockSpec((TM1, h2_pad), lambda i: (i, 0)),
            pl.BlockSpec((TM1, h2_pad), lambda i: (i, 0)),
        ),
        compiler_params=pltpu.CompilerParams(
            dimension_semantics=("parallel",)),
    )(x_p, w1_t, b1_r, w2_t, b2_r)

    hp = pl.pallas_call(
        _prop_kernel,
        out_shape=jax.ShapeDtypeStruct((N_pad, h2_pad), f32),
        grid=(N_pad // TM2,),
        in_specs=[
            pl.BlockSpec((TM2, N_pad // 2), lambda i: (i, 0)),
            pl.BlockSpec((TM2, N_pad // 2), lambda i: (i, 1)),
            pl.BlockSpec((N_pad, h2_pad), lambda i: (0, 0)),
        ],
        out_specs=pl.BlockSpec((TM2, h2_pad), lambda i: (i, 0)),
        compiler_params=pltpu.CompilerParams(
            dimension_semantics=("parallel",)),
        cost_estimate=pl.CostEstimate(
            flops=2 * N_pad * N_pad * h2_pad,
            transcendentals=0,
            bytes_accessed=4 * N_pad * N_pad + 2 * N_pad * h2_pad
                           + 4 * N_pad * h2_pad,
        ),
    )(adj_p, adj_p, ha16)

    return ha32[:N, :h2], hp[:N, :h2]


# speedup vs baseline: 1.2438x; 1.1150x over previous
import jax
import jax.numpy as jnp
from jax.experimental import pallas as pl
from jax.experimental.pallas import tpu as pltpu


def _round_up(x, m):
    return ((x + m - 1) // m) * m


def _fused_kernel(x_ref, w1_ref, b1_ref, w2_ref, b2_ref, adj_ref,
                  ha32_ref, hp_ref, ha16_ref):
    @pl.when(pl.program_id(0) == 0)
    def _():
        h = jnp.dot(x_ref[...], w1_ref[...], preferred_element_type=jnp.float32)
        h = jnp.maximum(h + b1_ref[...], 0.0)
        ha = jnp.dot(h, w2_ref[...], preferred_element_type=jnp.float32)
        ha = ha + b2_ref[...]
        ha32_ref[...] = ha
        ha16_ref[...] = ha.astype(jnp.bfloat16)

    a16 = adj_ref[...].astype(jnp.bfloat16)
    hp_ref[...] = jnp.dot(a16, ha16_ref[...],
                          preferred_element_type=jnp.float32)


def kernel(x, adj, w1, b1, w2, b2):
    N, n_in = x.shape
    h1 = w1.shape[0]
    h2 = w2.shape[0]
    f32 = jnp.float32

    h2_pad = _round_up(h2, 128)
    TM = 512 if N % 512 == 0 else 256
    N_pad = _round_up(N, TM)

    x_p = x.astype(f32)
    adj_p = adj.astype(f32)
    if N_pad != N:
        x_p = jnp.pad(x_p, ((0, N_pad - N), (0, 0)))
        adj_p = jnp.pad(adj_p, ((0, N_pad - N), (0, N_pad - N)))

    w1_t = w1.T.astype(f32)
    b1_r = b1.reshape(1, h1).astype(f32)
    w2_t = jnp.pad(w2.T.astype(f32), ((0, 0), (0, h2_pad - h2)))
    b2_r = jnp.pad(b2.reshape(1, h2).astype(f32), ((0, 0), (0, h2_pad - h2)))

    ha32, hp = pl.pallas_call(
        _fused_kernel,
        out_shape=(
            jax.ShapeDtypeStruct((N_pad, h2_pad), f32),
            jax.ShapeDtypeStruct((N_pad, h2_pad), f32),
        ),
        grid=(N_pad // TM,),
        in_specs=[
            pl.BlockSpec((N_pad, n_in), lambda i: (0, 0)),
            pl.BlockSpec((n_in, h1), lambda i: (0, 0)),
            pl.BlockSpec((1, h1), lambda i: (0, 0)),
            pl.BlockSpec((h1, h2_pad), lambda i: (0, 0)),
            pl.BlockSpec((1, h2_pad), lambda i: (0, 0)),
            pl.BlockSpec((TM, N_pad), lambda i: (i, 0)),
        ],
        out_specs=(
            pl.BlockSpec((N_pad, h2_pad), lambda i: (0, 0)),
            pl.BlockSpec((TM, h2_pad), lambda i: (i, 0)),
        ),
        scratch_shapes=[pltpu.VMEM((N_pad, h2_pad), jnp.bfloat16)],
        compiler_params=pltpu.CompilerParams(
            dimension_semantics=("arbitrary",)),
        cost_estimate=pl.CostEstimate(
            flops=2 * N_pad * N_pad * h2_pad
                  + 2 * N_pad * n_in * h1 + 2 * N_pad * h1 * h2_pad,
            transcendentals=0,
            bytes_accessed=4 * N_pad * N_pad + 4 * N_pad * n_in
                           + 8 * N_pad * h2_pad,
        ),
    )(x_p, w1_t, b1_r, w2_t, b2_r, adj_p)

    return ha32[:N, :h2], hp[:N, :h2]


# speedup vs baseline: 1.2525x; 1.0070x over previous
import jax
import jax.numpy as jnp
from jax.experimental import pallas as pl
from jax.experimental.pallas import tpu as pltpu


def _round_up(x, m):
    return ((x + m - 1) // m) * m


def _fused_kernel(x_ref, w1_ref, b1_ref, w2_ref, b2_ref, adj_ref,
                  ha32_ref, hp_ref, ha16_ref):
    @pl.when(pl.program_id(0) == 0)
    def _():
        h = jnp.dot(x_ref[...], w1_ref[...], preferred_element_type=jnp.float32)
        h = jnp.maximum(h + b1_ref[...], 0.0)
        ha = jnp.dot(h, w2_ref[...], preferred_element_type=jnp.float32)
        ha = ha + b2_ref[...]
        ha32_ref[...] = ha
        ha16_ref[...] = ha.astype(jnp.bfloat16)

    i = pl.program_id(0)
    tm = adj_ref.shape[0]
    a16 = adj_ref[...].astype(jnp.bfloat16)
    hp_ref[pl.ds(i * tm, tm), :] = jnp.dot(
        a16, ha16_ref[...], preferred_element_type=jnp.float32)


def kernel(x, adj, w1, b1, w2, b2):
    N, n_in = x.shape
    h1 = w1.shape[0]
    h2 = w2.shape[0]
    f32 = jnp.float32

    h2_pad = _round_up(h2, 128)
    TM = 512 if N % 512 == 0 else 256
    N_pad = _round_up(N, TM)

    x_p = x.astype(f32)
    adj_p = adj.astype(f32)
    if N_pad != N:
        x_p = jnp.pad(x_p, ((0, N_pad - N), (0, 0)))
        adj_p = jnp.pad(adj_p, ((0, N_pad - N), (0, N_pad - N)))

    w1_t = w1.T.astype(f32)
    b1_r = b1.reshape(1, h1).astype(f32)
    w2_t = jnp.pad(w2.T.astype(f32), ((0, 0), (0, h2_pad - h2)))
    b2_r = jnp.pad(b2.reshape(1, h2).astype(f32), ((0, 0), (0, h2_pad - h2)))

    ha32, hp = pl.pallas_call(
        _fused_kernel,
        out_shape=(
            jax.ShapeDtypeStruct((N_pad, h2_pad), f32),
            jax.ShapeDtypeStruct((N_pad, h2_pad), f32),
        ),
        grid=(N_pad // TM,),
        in_specs=[
            pl.BlockSpec((N_pad, n_in), lambda i: (0, 0)),
            pl.BlockSpec((n_in, h1), lambda i: (0, 0)),
            pl.BlockSpec((1, h1), lambda i: (0, 0)),
            pl.BlockSpec((h1, h2_pad), lambda i: (0, 0)),
            pl.BlockSpec((1, h2_pad), lambda i: (0, 0)),
            pl.BlockSpec((TM, N_pad), lambda i: (i, 0)),
        ],
        out_specs=(
            pl.BlockSpec((N_pad, h2_pad), lambda i: (0, 0)),
            pl.BlockSpec((N_pad, h2_pad), lambda i: (0, 0)),
        ),
        scratch_shapes=[pltpu.VMEM((N_pad, h2_pad), jnp.bfloat16)],
        compiler_params=pltpu.CompilerParams(
            dimension_semantics=("arbitrary",)),
        cost_estimate=pl.CostEstimate(
            flops=2 * N_pad * N_pad * h2_pad
                  + 2 * N_pad * n_in * h1 + 2 * N_pad * h1 * h2_pad,
            transcendentals=0,
            bytes_accessed=4 * N_pad * N_pad + 4 * N_pad * n_in
                           + 8 * N_pad * h2_pad,
        ),
    )(x_p, w1_t, b1_r, w2_t, b2_r, adj_p)

    return ha32[:N, :h2], hp[:N, :h2]


# speedup vs baseline: 1.4166x; 1.1310x over previous
import jax
import jax.numpy as jnp
from jax.experimental import pallas as pl
from jax.experimental.pallas import tpu as pltpu

_TRANS_RHS = (((1,), (1,)), ((), ()))


def _round_up(x, m):
    return ((x + m - 1) // m) * m


def _fused_kernel(x_ref, w1_ref, b1_ref, w2_ref, b2_ref, adj_ref,
                  ha32_ref, hp_ref, ha16_ref):
    @pl.when(pl.program_id(0) == 0)
    def _():
        h = jax.lax.dot_general(x_ref[...], w1_ref[...], _TRANS_RHS,
                                preferred_element_type=jnp.float32)
        h = jnp.maximum(h + b1_ref[...], 0.0)
        ha = jax.lax.dot_general(h, w2_ref[...], _TRANS_RHS,
                                 preferred_element_type=jnp.float32)
        ha = ha + b2_ref[...]
        ha32_ref[...] = ha
        ha16_ref[...] = ha.astype(jnp.bfloat16)

    i = pl.program_id(0)
    tm = adj_ref.shape[0]
    a16 = adj_ref[...].astype(jnp.bfloat16)
    hp_ref[pl.ds(i * tm, tm), :] = jnp.dot(
        a16, ha16_ref[...], preferred_element_type=jnp.float32)


def kernel(x, adj, w1, b1, w2, b2):
    N, n_in = x.shape
    h1 = w1.shape[0]
    h2 = w2.shape[0]
    f32 = jnp.float32

    TM = 512 if N % 512 == 0 else 256
    N_pad = _round_up(N, TM)

    x_p = x.astype(f32)
    adj_p = adj.astype(f32)
    if N_pad != N:
        x_p = jnp.pad(x_p, ((0, N_pad - N), (0, 0)))
        adj_p = jnp.pad(adj_p, ((0, N_pad - N), (0, N_pad - N)))
    b1_r = b1.reshape(1, h1).astype(f32)
    b2_r = b2.reshape(1, h2).astype(f32)

    ha32, hp = pl.pallas_call(
        _fused_kernel,
        out_shape=(
            jax.ShapeDtypeStruct((N_pad, h2), f32),
            jax.ShapeDtypeStruct((N_pad, h2), f32),
        ),
        grid=(N_pad // TM,),
        in_specs=[
            pl.BlockSpec((N_pad, n_in), lambda i: (0, 0)),
            pl.BlockSpec((h1, n_in), lambda i: (0, 0)),
            pl.BlockSpec((1, h1), lambda i: (0, 0)),
            pl.BlockSpec((h2, h1), lambda i: (0, 0)),
            pl.BlockSpec((1, h2), lambda i: (0, 0)),
            pl.BlockSpec((TM, N_pad), lambda i: (i, 0)),
        ],
        out_specs=(
            pl.BlockSpec((N_pad, h2), lambda i: (0, 0)),
            pl.BlockSpec((N_pad, h2), lambda i: (0, 0)),
        ),
        scratch_shapes=[pltpu.VMEM((N_pad, h2), jnp.bfloat16)],
        compiler_params=pltpu.CompilerParams(
            dimension_semantics=("arbitrary",)),
        cost_estimate=pl.CostEstimate(
            flops=2 * N_pad * N_pad * h2
                  + 2 * N_pad * n_in * h1 + 2 * N_pad * h1 * h2,
            transcendentals=0,
            bytes_accessed=4 * N_pad * N_pad + 4 * N_pad * n_in
                           + 8 * N_pad * h2,
        ),
    )(x_p, w1, b1_r, w2, b2_r, adj_p)

    return ha32[:N, :h2], hp[:N, :h2]
